# speedup vs baseline: 2.6310x; 2.6310x over previous
import functools

import jax
import jax.numpy as jnp
from jax import lax
from jax.experimental import pallas as pl
from jax.experimental.pallas import tpu as pltpu
from jax.experimental.pallas import tpu_sc as plsc

NC = 2
NS = 16
CW = 128
EB = 128
GG = 64
HD = 512
BLK = 512


@functools.cache
def _edge_agg_fn(n_chunks, n_pad, e_pad):
    cpc = n_chunks // NC
    per_sub = e_pad // NS
    nb = per_sub // EB
    rows_per_sub = n_pad // NS
    mesh = plsc.VectorSubcoreMesh(core_axis_name="c", subcore_axis_name="s",
                                  num_cores=NC, num_subcores=NS)

    def body(src_hbm, dst_hbm, zeros_hbm, *refs):
        h_refs = refs[:n_chunks]
        out_refs = refs[n_chunks:2 * n_chunks]
        src_v, dst_v, rows_v, accum, sem = refs[2 * n_chunks:]
        c = lax.axis_index("c")
        s = lax.axis_index("s")
        r0 = s * rows_per_sub

        def process(h_hbm, out_hbm):
            pltpu.sync_copy(zeros_hbm.at[pl.ds(r0, rows_per_sub)],
                            accum.at[pl.ds(r0, rows_per_sub)])
            plsc.subcore_barrier()

            def step(i, carry):
                e0 = s * per_sub + i * EB
                pltpu.sync_copy(src_hbm.at[pl.ds(e0, EB)], src_v)
                pltpu.sync_copy(dst_hbm.at[pl.ds(e0, EB)], dst_v)
                pltpu.async_copy(h_hbm.at[src_v], rows_v, sem).wait()
                pltpu.sync_copy(rows_v, accum.at[dst_v], add=True)
                return carry

            lax.fori_loop(0, nb, step, 0)
            plsc.subcore_barrier()
            pltpu.sync_copy(accum.at[pl.ds(r0, rows_per_sub)],
                            out_hbm.at[pl.ds(r0, rows_per_sub)])

        for k in range(cpc):
            @pl.when(c == 0)
            def _(k=k):
                process(h_refs[k], out_refs[k])

            @pl.when(c == 1)
            def _(k=k):
                process(h_refs[cpc + k], out_refs[cpc + k])

    return pl.kernel(
        body,
        out_type=[jax.ShapeDtypeStruct((n_pad, CW), jnp.float32)] * n_chunks,
        mesh=mesh,
        scratch_types=[
            pltpu.VMEM((EB,), jnp.int32),
            pltpu.VMEM((EB,), jnp.int32),
            pltpu.VMEM((EB, CW), jnp.float32),
            pltpu.VMEM_SHARED((n_pad, CW), jnp.float32),
            pltpu.SemaphoreType.DMA,
        ],
    )


@functools.cache
def _mlp_pool_fn(fin, n_chunks, n_pad):
    grid = n_pad // BLK

    def body(*refs):
        h_ref = refs[0]
        a_refs = refs[1:1 + n_chunks]
        (batch_ref, w1_ref, w2_ref, b1_ref, b2_ref,
         hout_ref, pooled_ref) = refs[1 + n_chunks:]
        i = pl.program_id(0)
        agg = jnp.concatenate([a[...] for a in a_refs], axis=1)
        hin = h_ref[...] + agg
        t = jnp.dot(hin, w1_ref[...], preferred_element_type=jnp.float32)
        t = jnp.maximum(t + b1_ref[0:1, :], 0.0)
        o = jnp.dot(t, w2_ref[...], preferred_element_type=jnp.float32)
        o = jnp.maximum(o + b2_ref[0:1, :], 0.0)
        b = batch_ref[0, 0, :]
        o = jnp.where(b.reshape(BLK, 1) < GG, o, 0.0)
        hout_ref[...] = o
        onehot = (jax.lax.broadcasted_iota(jnp.int32, (GG, BLK), 0)
                  == b.reshape(1, BLK)).astype(jnp.float32)
        ppart = jnp.dot(onehot, o, preferred_element_type=jnp.float32)

        @pl.when(i == 0)
        def _():
            pooled_ref[...] = ppart

        @pl.when(i > 0)
        def _():
            pooled_ref[...] += ppart

    in_specs = [pl.BlockSpec((BLK, fin), lambda i: (i, 0))]
    in_specs += [pl.BlockSpec((BLK, CW), lambda i: (i, 0))] * n_chunks
    in_specs += [
        pl.BlockSpec((1, 1, BLK), lambda i: (i, 0, 0)),
        pl.BlockSpec((fin, HD), lambda i: (0, 0)),
        pl.BlockSpec((HD, HD), lambda i: (0, 0)),
        pl.BlockSpec((8, HD), lambda i: (0, 0)),
        pl.BlockSpec((8, HD), lambda i: (0, 0)),
    ]
    out_specs = [
        pl.BlockSpec((BLK, HD), lambda i: (i, 0)),
        pl.BlockSpec((GG, HD), lambda i: (0, 0)),
    ]
    return pl.pallas_call(
        body,
        grid=(grid,),
        in_specs=in_specs,
        out_specs=out_specs,
        out_shape=[jax.ShapeDtypeStruct((n_pad, HD), jnp.float32),
                   jax.ShapeDtypeStruct((GG, HD), jnp.float32)],
    )


def kernel(x, edge_index, batch, W1_0, b1_0, W2_0, b2_0, W1_1, b1_1, W2_1,
           b2_1, W1_2, b1_2, W2_2, b2_2):
    n, _ = x.shape
    e = edge_index.shape[1]
    n_pad = ((n + BLK - 1) // BLK) * BLK
    e_unit = NS * EB
    e_pad = ((e + e_unit - 1) // e_unit) * e_unit
    src = jnp.pad(edge_index[0], (0, e_pad - e), constant_values=n)
    dst = jnp.pad(edge_index[1], (0, e_pad - e), constant_values=n)
    h = jnp.pad(x, ((0, n_pad - n), (0, 0)))
    batch_p = jnp.pad(batch, (0, n_pad - n), constant_values=GG)
    batch3d = batch_p.reshape(n_pad // BLK, 1, BLK)
    zeros_tab = jnp.zeros((n_pad, CW), jnp.float32)

    params = [(W1_0, b1_0, W2_0, b2_0), (W1_1, b1_1, W2_1, b2_1),
              (W1_2, b1_2, W2_2, b2_2)]
    pooled_all = []
    for (W1, b1, W2, b2) in params:
        fin = W1.shape[0]
        nch = fin // CW
        chunks = [lax.slice_in_dim(h, k * CW, (k + 1) * CW, axis=1)
                  for k in range(nch)]
        aggs = _edge_agg_fn(nch, n_pad, e_pad)(src, dst, zeros_tab, *chunks)
        b1b = jnp.broadcast_to(b1.reshape(1, HD), (8, HD))
        b2b = jnp.broadcast_to(b2.reshape(1, HD), (8, HD))
        h, pooled = _mlp_pool_fn(fin, nch, n_pad)(
            h, *aggs, batch3d, W1, W2, b1b, b2b)
        pooled_all.append(pooled)

    z = jnp.concatenate(pooled_all, axis=1)
    center = jnp.zeros((1, HD * len(params)), jnp.float32)
    return (z, center)


# speedup vs baseline: 2.9172x; 1.1088x over previous
import functools

import jax
import jax.numpy as jnp
from jax import lax
from jax.experimental import pallas as pl
from jax.experimental.pallas import tpu as pltpu
from jax.experimental.pallas import tpu_sc as plsc

NC = 2
NS = 16
CW = 128
EB = 128
GG = 64
HD = 512
BLK = 512


KB = 16
NBUF = 2
LOOK = 2


@functools.cache
def _edge_agg_fn(n_chunks, n_pad, e_pad):
    cpc = n_chunks // NC
    per_sub = e_pad // NS
    nb_super = per_sub // (KB * EB)
    idx_rows_per_sub = per_sub // EB
    rows_per_sub = n_pad // NS
    mesh = plsc.VectorSubcoreMesh(core_axis_name="c", subcore_axis_name="s",
                                  num_cores=NC, num_subcores=NS)

    def body(src_hbm, dst_hbm, zeros_hbm, *refs):
        h_refs = refs[:n_chunks]
        out_refs = refs[n_chunks:2 * n_chunks]
        rest = refs[2 * n_chunks:]
        src_i, dst_i = rest[0], rest[1]
        rows = rest[2:2 + NBUF]
        accum, gsem, ssem = rest[2 + NBUF:]
        c = lax.axis_index("c")
        s = lax.axis_index("s")
        r0 = s * rows_per_sub

        def process(h_hbm, out_hbm):
            pltpu.sync_copy(zeros_hbm.at[pl.ds(r0, rows_per_sub)],
                            accum.at[pl.ds(r0, rows_per_sub)])
            plsc.subcore_barrier()

            def step(b, carry):
                ri = s * idx_rows_per_sub + b * KB
                pltpu.sync_copy(src_hbm.at[pl.ds(ri, KB)], src_i)
                pltpu.sync_copy(dst_hbm.at[pl.ds(ri, KB)], dst_i)
                gd = [None] * KB
                sd = [None] * KB
                for j in range(LOOK):
                    gd[j] = pltpu.async_copy(
                        h_hbm.at[src_i.at[j]], rows[j % NBUF], gsem)
                for j in range(KB):
                    gd[j].wait()
                    sd[j] = pltpu.async_copy(
                        rows[j % NBUF], accum.at[dst_i.at[j]], ssem, add=True)
                    nxt = j + LOOK
                    if nxt < KB:
                        if nxt - NBUF >= 0:
                            sd[nxt - NBUF].wait()
                        gd[nxt] = pltpu.async_copy(
                            h_hbm.at[src_i.at[nxt]], rows[nxt % NBUF], gsem)
                for j in range(KB - NBUF, KB):
                    sd[j].wait()
                return carry

            lax.fori_loop(0, nb_super, step, 0)
            plsc.subcore_barrier()
            pltpu.sync_copy(accum.at[pl.ds(r0, rows_per_sub)],
                            out_hbm.at[pl.ds(r0, rows_per_sub)])

        for k in range(cpc):
            @pl.when(c == 0)
            def _(k=k):
                process(h_refs[k], out_refs[k])

            @pl.when(c == 1)
            def _(k=k):
                process(h_refs[cpc + k], out_refs[cpc + k])

    return pl.kernel(
        body,
        out_type=[jax.ShapeDtypeStruct((n_pad, CW), jnp.float32)] * n_chunks,
        mesh=mesh,
        scratch_types=(
            [pltpu.VMEM((KB, EB), jnp.int32)] * 2
            + [pltpu.VMEM((EB, CW), jnp.float32)] * NBUF
            + [pltpu.VMEM_SHARED((n_pad, CW), jnp.float32),
               pltpu.SemaphoreType.DMA,
               pltpu.SemaphoreType.DMA]
        ),
    )


@functools.cache
def _mlp_pool_fn(fin, n_chunks, n_pad):
    grid = n_pad // BLK

    def body(*refs):
        h_ref = refs[0]
        a_refs = refs[1:1 + n_chunks]
        (batch_ref, w1_ref, w2_ref, b1_ref, b2_ref,
         hout_ref, pooled_ref) = refs[1 + n_chunks:]
        i = pl.program_id(0)
        agg = jnp.concatenate([a[...] for a in a_refs], axis=1)
        hin = h_ref[...] + agg
        t = jnp.dot(hin, w1_ref[...], preferred_element_type=jnp.float32)
        t = jnp.maximum(t + b1_ref[0:1, :], 0.0)
        o = jnp.dot(t, w2_ref[...], preferred_element_type=jnp.float32)
        o = jnp.maximum(o + b2_ref[0:1, :], 0.0)
        b = batch_ref[0, 0, :]
        o = jnp.where(b.reshape(BLK, 1) < GG, o, 0.0)
        hout_ref[...] = o
        onehot = (jax.lax.broadcasted_iota(jnp.int32, (GG, BLK), 0)
                  == b.reshape(1, BLK)).astype(jnp.float32)
        ppart = jnp.dot(onehot, o, preferred_element_type=jnp.float32)

        @pl.when(i == 0)
        def _():
            pooled_ref[...] = ppart

        @pl.when(i > 0)
        def _():
            pooled_ref[...] += ppart

    in_specs = [pl.BlockSpec((BLK, fin), lambda i: (i, 0))]
    in_specs += [pl.BlockSpec((BLK, CW), lambda i: (i, 0))] * n_chunks
    in_specs += [
        pl.BlockSpec((1, 1, BLK), lambda i: (i, 0, 0)),
        pl.BlockSpec((fin, HD), lambda i: (0, 0)),
        pl.BlockSpec((HD, HD), lambda i: (0, 0)),
        pl.BlockSpec((8, HD), lambda i: (0, 0)),
        pl.BlockSpec((8, HD), lambda i: (0, 0)),
    ]
    out_specs = [
        pl.BlockSpec((BLK, HD), lambda i: (i, 0)),
        pl.BlockSpec((GG, HD), lambda i: (0, 0)),
    ]
    return pl.pallas_call(
        body,
        grid=(grid,),
        in_specs=in_specs,
        out_specs=out_specs,
        out_shape=[jax.ShapeDtypeStruct((n_pad, HD), jnp.float32),
                   jax.ShapeDtypeStruct((GG, HD), jnp.float32)],
    )


def kernel(x, edge_index, batch, W1_0, b1_0, W2_0, b2_0, W1_1, b1_1, W2_1,
           b2_1, W1_2, b1_2, W2_2, b2_2):
    n, _ = x.shape
    e = edge_index.shape[1]
    n_pad = ((n + BLK - 1) // BLK) * BLK
    e_unit = NS * KB * EB
    e_pad = ((e + e_unit - 1) // e_unit) * e_unit
    src = jnp.pad(edge_index[0], (0, e_pad - e),
                  constant_values=n).reshape(e_pad // EB, EB)
    dst = jnp.pad(edge_index[1], (0, e_pad - e),
                  constant_values=n).reshape(e_pad // EB, EB)
    h = jnp.pad(x, ((0, n_pad - n), (0, 0)))
    batch_p = jnp.pad(batch, (0, n_pad - n), constant_values=GG)
    batch3d = batch_p.reshape(n_pad // BLK, 1, BLK)
    zeros_tab = jnp.zeros((n_pad, CW), jnp.float32)

    params = [(W1_0, b1_0, W2_0, b2_0), (W1_1, b1_1, W2_1, b2_1),
              (W1_2, b1_2, W2_2, b2_2)]
    pooled_all = []
    for (W1, b1, W2, b2) in params:
        fin = W1.shape[0]
        nch = fin // CW
        chunks = [lax.slice_in_dim(h, k * CW, (k + 1) * CW, axis=1)
                  for k in range(nch)]
        aggs = _edge_agg_fn(nch, n_pad, e_pad)(src, dst, zeros_tab, *chunks)
        b1b = jnp.broadcast_to(b1.reshape(1, HD), (8, HD))
        b2b = jnp.broadcast_to(b2.reshape(1, HD), (8, HD))
        h, pooled = _mlp_pool_fn(fin, nch, n_pad)(
            h, *aggs, batch3d, W1, W2, b1b, b2b)
        pooled_all.append(pooled)

    z = jnp.concatenate(pooled_all, axis=1)
    center = jnp.zeros((1, HD * len(params)), jnp.float32)
    return (z, center)


# speedup vs baseline: 4.2147x; 1.4448x over previous
import functools

import jax
import jax.numpy as jnp
from jax import lax
from jax.experimental import pallas as pl
from jax.experimental.pallas import tpu as pltpu
from jax.experimental.pallas import tpu_sc as plsc

NC = 2
NS = 16
CW = 256
EB = 128
GG = 64
HD = 512
BLK = 512
NBUF = 2
LOOK = 2


@functools.cache
def _edge_agg_fn(n_pad, e_pad, edge_split, kb):
    e_core = e_pad // NC if edge_split else e_pad
    per_sub = e_core // NS
    nb_super = per_sub // (kb * EB)
    idx_rows_per_sub = per_sub // EB
    rows_per_sub = n_pad // NS
    mesh = plsc.VectorSubcoreMesh(core_axis_name="c", subcore_axis_name="s",
                                  num_cores=NC, num_subcores=NS)

    def body(src_hbm, dst_hbm, zeros_hbm, *refs):
        if edge_split:
            h_refs = (refs[0], refs[0])
            out_refs = refs[1:3]
            rest = refs[3:]
        else:
            h_refs = refs[0:2]
            out_refs = refs[2:4]
            rest = refs[4:]
        src_i, dst_i = rest[0], rest[1]
        rows = rest[2:2 + NBUF]
        accum, gsem, ssem = rest[2 + NBUF:]
        c = lax.axis_index("c")
        s = lax.axis_index("s")
        r0 = s * rows_per_sub
        core_idx0 = c * (idx_rows_per_sub * NS) if edge_split else 0

        def process(h_hbm, out_hbm):
            pltpu.sync_copy(zeros_hbm.at[pl.ds(r0, rows_per_sub)],
                            accum.at[pl.ds(r0, rows_per_sub)])
            plsc.subcore_barrier()

            def step(b, carry):
                ri = core_idx0 + s * idx_rows_per_sub + b * kb
                pltpu.sync_copy(src_hbm.at[pl.ds(ri, kb)], src_i)
                pltpu.sync_copy(dst_hbm.at[pl.ds(ri, kb)], dst_i)
                gd = [None] * kb
                sd = [None] * kb
                for j in range(LOOK):
                    gd[j] = pltpu.async_copy(
                        h_hbm.at[src_i.at[j]], rows[j % NBUF], gsem)
                for j in range(kb):
                    gd[j].wait()
                    sd[j] = pltpu.async_copy(
                        rows[j % NBUF], accum.at[dst_i.at[j]], ssem,
                        add=True)
                    nxt = j + LOOK
                    if nxt < kb:
                        if nxt - NBUF >= 0:
                            sd[nxt - NBUF].wait()
                        gd[nxt] = pltpu.async_copy(
                            h_hbm.at[src_i.at[nxt]], rows[nxt % NBUF], gsem)
                for j in range(kb - NBUF, kb):
                    sd[j].wait()
                return carry

            lax.fori_loop(0, nb_super, step, 0)
            plsc.subcore_barrier()
            pltpu.sync_copy(accum.at[pl.ds(r0, rows_per_sub)],
                            out_hbm.at[pl.ds(r0, rows_per_sub)])

        @pl.when(c == 0)
        def _():
            process(h_refs[0], out_refs[0])

        @pl.when(c == 1)
        def _():
            process(h_refs[1], out_refs[1])

    return pl.kernel(
        body,
        out_type=[jax.ShapeDtypeStruct((n_pad, 2, 128), jnp.bfloat16)] * 2,
        mesh=mesh,
        compiler_params=pltpu.CompilerParams(use_tc_tiling_on_sc=False),
        scratch_types=(
            [pltpu.VMEM((kb, EB), jnp.int32)] * 2
            + [pltpu.VMEM((EB, 2, 128), jnp.bfloat16)] * NBUF
            + [pltpu.VMEM_SHARED((n_pad, 2, 128), jnp.bfloat16),
               pltpu.SemaphoreType.DMA,
               pltpu.SemaphoreType.DMA]
        ),
    )


@functools.cache
def _mlp_pool_fn(fin, agg_sum, n_pad):
    grid = n_pad // BLK

    def body(*refs):
        (h_ref, a0_ref, a1_ref, batch_ref, w1_ref, w2_ref, b1_ref, b2_ref,
         hout_ref, hbf_ref, pooled_ref) = refs
        i = pl.program_id(0)
        a0 = a0_ref[...].astype(jnp.float32)
        a1 = a1_ref[...].astype(jnp.float32)
        if agg_sum:
            agg = a0 + a1
        else:
            agg = jnp.concatenate([a0, a1], axis=1)
        hin = h_ref[...] + agg
        t = jnp.dot(hin, w1_ref[...], preferred_element_type=jnp.float32)
        t = jnp.maximum(t + b1_ref[0:1, :], 0.0)
        o = jnp.dot(t, w2_ref[...], preferred_element_type=jnp.float32)
        o = jnp.maximum(o + b2_ref[0:1, :], 0.0)
        b = batch_ref[0, 0, :]
        o = jnp.where(b.reshape(BLK, 1) < GG, o, 0.0)
        hout_ref[...] = o
        hbf_ref[...] = o.astype(jnp.bfloat16)
        onehot = (jax.lax.broadcasted_iota(jnp.int32, (GG, BLK), 0)
                  == b.reshape(1, BLK)).astype(jnp.float32)
        ppart = jnp.dot(onehot, o, preferred_element_type=jnp.float32)

        @pl.when(i == 0)
        def _():
            pooled_ref[...] = ppart

        @pl.when(i > 0)
        def _():
            pooled_ref[...] += ppart

    in_specs = [
        pl.BlockSpec((BLK, fin), lambda i: (i, 0)),
        pl.BlockSpec((BLK, CW), lambda i: (i, 0)),
        pl.BlockSpec((BLK, CW), lambda i: (i, 0)),
        pl.BlockSpec((1, 1, BLK), lambda i: (i, 0, 0)),
        pl.BlockSpec((fin, HD), lambda i: (0, 0)),
        pl.BlockSpec((HD, HD), lambda i: (0, 0)),
        pl.BlockSpec((8, HD), lambda i: (0, 0)),
        pl.BlockSpec((8, HD), lambda i: (0, 0)),
    ]
    out_specs = [
        pl.BlockSpec((BLK, HD), lambda i: (i, 0)),
        pl.BlockSpec((BLK, HD), lambda i: (i, 0)),
        pl.BlockSpec((GG, HD), lambda i: (0, 0)),
    ]
    return pl.pallas_call(
        body,
        grid=(grid,),
        in_specs=in_specs,
        out_specs=out_specs,
        out_shape=[jax.ShapeDtypeStruct((n_pad, HD), jnp.float32),
                   jax.ShapeDtypeStruct((n_pad, HD), jnp.bfloat16),
                   jax.ShapeDtypeStruct((GG, HD), jnp.float32)],
    )


def kernel(x, edge_index, batch, W1_0, b1_0, W2_0, b2_0, W1_1, b1_1, W2_1,
           b2_1, W1_2, b1_2, W2_2, b2_2):
    n, fin0 = x.shape
    e = edge_index.shape[1]
    n_pad = ((n + BLK - 1) // BLK) * BLK
    e_unit = NS * 16 * EB
    e_pad = ((e + e_unit - 1) // e_unit) * e_unit
    src = jnp.pad(edge_index[0], (0, e_pad - e),
                  constant_values=n).reshape(e_pad // EB, EB)
    dst = jnp.pad(edge_index[1], (0, e_pad - e),
                  constant_values=n).reshape(e_pad // EB, EB)
    h = jnp.pad(x, ((0, n_pad - n), (0, 0)))
    h_bf = h.astype(jnp.bfloat16)
    batch_p = jnp.pad(batch, (0, n_pad - n), constant_values=GG)
    batch3d = batch_p.reshape(n_pad // BLK, 1, BLK)
    zeros_tab = jnp.zeros((n_pad, 2, 128), jnp.bfloat16)

    params = [(W1_0, b1_0, W2_0, b2_0), (W1_1, b1_1, W2_1, b2_1),
              (W1_2, b1_2, W2_2, b2_2)]
    pooled_all = []
    for (W1, b1, W2, b2) in params:
        fin = W1.shape[0]
        edge_split = fin == CW
        if edge_split:
            a0, a1 = _edge_agg_fn(n_pad, e_pad, True, 8)(
                src, dst, zeros_tab, h_bf.reshape(n_pad, 2, 128))
        else:
            a0, a1 = _edge_agg_fn(n_pad, e_pad, False, 16)(
                src, dst, zeros_tab,
                lax.slice_in_dim(h_bf, 0, CW, axis=1).reshape(n_pad, 2, 128),
                lax.slice_in_dim(h_bf, CW, 2 * CW,
                                 axis=1).reshape(n_pad, 2, 128))
        a0 = a0.reshape(n_pad, CW)
        a1 = a1.reshape(n_pad, CW)
        b1b = jnp.broadcast_to(b1.reshape(1, HD), (8, HD))
        b2b = jnp.broadcast_to(b2.reshape(1, HD), (8, HD))
        h, h_bf, pooled = _mlp_pool_fn(fin, edge_split, n_pad)(
            h, a0, a1, batch3d, W1, W2, b1b, b2b)
        pooled_all.append(pooled)

    z = jnp.concatenate(pooled_all, axis=1)
    center = jnp.zeros((1, HD * len(params)), jnp.float32)
    return (z, center)


# speedup vs baseline: 4.2459x; 1.0074x over previous
import functools

import jax
import jax.numpy as jnp
from jax import lax
from jax.experimental import pallas as pl
from jax.experimental.pallas import tpu as pltpu
from jax.experimental.pallas import tpu_sc as plsc

NC = 2
NS = 16
CW = 256
EB = 128
GG = 64
HD = 512
BLK = 512
NBUF = 2
LOOK = 2


@functools.cache
def _edge_agg_fn(n_pad, e_pad, edge_split, kb):
    e_core = e_pad // NC if edge_split else e_pad
    per_sub = e_core // NS
    nb_super = per_sub // (kb * EB)
    idx_rows_per_sub = per_sub // EB
    rows_per_sub = n_pad // NS
    mesh = plsc.VectorSubcoreMesh(core_axis_name="c", subcore_axis_name="s",
                                  num_cores=NC, num_subcores=NS)

    def body(src_hbm, dst_hbm, zeros_hbm, *refs):
        if edge_split:
            h_refs = (refs[0], refs[0])
            out_refs = refs[1:3]
            rest = refs[3:]
        else:
            h_refs = refs[0:2]
            out_refs = refs[2:4]
            rest = refs[4:]
        src_i, dst_i = rest[0], rest[1]
        rows = rest[2:2 + NBUF]
        accum, gsem, ssem = rest[2 + NBUF:]
        c = lax.axis_index("c")
        s = lax.axis_index("s")
        r0 = s * rows_per_sub
        core_idx0 = c * (idx_rows_per_sub * NS) if edge_split else 0

        def process(h_hbm, out_hbm):
            pltpu.sync_copy(zeros_hbm.at[pl.ds(r0, rows_per_sub)],
                            accum.at[pl.ds(r0, rows_per_sub)])
            plsc.subcore_barrier()

            def step(b, carry):
                ri = core_idx0 + s * idx_rows_per_sub + b * kb
                pltpu.sync_copy(src_hbm.at[pl.ds(ri, kb)], src_i)
                pltpu.sync_copy(dst_hbm.at[pl.ds(ri, kb)], dst_i)
                gd = [None] * kb
                sd = [None] * kb
                for j in range(LOOK):
                    gd[j] = pltpu.async_copy(
                        h_hbm.at[src_i.at[j]], rows[j % NBUF], gsem)
                for j in range(kb):
                    gd[j].wait()
                    sd[j] = pltpu.async_copy(
                        rows[j % NBUF], accum.at[dst_i.at[j]], ssem,
                        add=True)
                    nxt = j + LOOK
                    if nxt < kb:
                        if nxt - NBUF >= 0:
                            sd[nxt - NBUF].wait()
                        gd[nxt] = pltpu.async_copy(
                            h_hbm.at[src_i.at[nxt]], rows[nxt % NBUF], gsem)
                for j in range(kb - NBUF, kb):
                    sd[j].wait()
                return carry

            lax.fori_loop(0, nb_super, step, 0)
            plsc.subcore_barrier()
            pltpu.sync_copy(accum.at[pl.ds(r0, rows_per_sub)],
                            out_hbm.at[pl.ds(r0, rows_per_sub)])

        @pl.when(c == 0)
        def _():
            process(h_refs[0], out_refs[0])

        @pl.when(c == 1)
        def _():
            process(h_refs[1], out_refs[1])

    return pl.kernel(
        body,
        out_type=[jax.ShapeDtypeStruct((n_pad, 2, 128), jnp.bfloat16)] * 2,
        mesh=mesh,
        compiler_params=pltpu.CompilerParams(use_tc_tiling_on_sc=False),
        scratch_types=(
            [pltpu.VMEM((kb, EB), jnp.int32)] * 2
            + [pltpu.VMEM((EB, 2, 128), jnp.bfloat16)] * NBUF
            + [pltpu.VMEM_SHARED((n_pad, 2, 128), jnp.bfloat16),
               pltpu.SemaphoreType.DMA,
               pltpu.SemaphoreType.DMA]
        ),
    )


@functools.cache
def _mlp_pool_fn(fin, agg_sum, n_pad):
    grid = n_pad // BLK

    def body(*refs):
        (h_ref, a0_ref, a1_ref, batch_ref, w1_ref, w2_ref, b1_ref, b2_ref,
         hout_ref, hbf_ref, pooled_ref) = refs
        i = pl.program_id(0)
        a0 = a0_ref[...].astype(jnp.float32)
        a1 = a1_ref[...].astype(jnp.float32)
        if agg_sum:
            agg = a0 + a1
        else:
            agg = jnp.concatenate([a0, a1], axis=1)
        hin = (h_ref[...] + agg).astype(jnp.bfloat16)
        t = jnp.dot(hin, w1_ref[...], preferred_element_type=jnp.float32)
        t = jnp.maximum(t + b1_ref[0:1, :], 0.0).astype(jnp.bfloat16)
        o = jnp.dot(t, w2_ref[...], preferred_element_type=jnp.float32)
        o = jnp.maximum(o + b2_ref[0:1, :], 0.0)
        b = batch_ref[0, 0, :]
        o = jnp.where(b.reshape(BLK, 1) < GG, o, 0.0)
        hout_ref[...] = o
        hbf_ref[...] = o.astype(jnp.bfloat16)
        onehot = (jax.lax.broadcasted_iota(jnp.int32, (GG, BLK), 0)
                  == b.reshape(1, BLK)).astype(jnp.float32)
        ppart = jnp.dot(onehot, o, preferred_element_type=jnp.float32)

        @pl.when(i == 0)
        def _():
            pooled_ref[...] = ppart

        @pl.when(i > 0)
        def _():
            pooled_ref[...] += ppart

    in_specs = [
        pl.BlockSpec((BLK, fin), lambda i: (i, 0)),
        pl.BlockSpec((BLK, CW), lambda i: (i, 0)),
        pl.BlockSpec((BLK, CW), lambda i: (i, 0)),
        pl.BlockSpec((1, 1, BLK), lambda i: (i, 0, 0)),
        pl.BlockSpec((fin, HD), lambda i: (0, 0)),
        pl.BlockSpec((HD, HD), lambda i: (0, 0)),
        pl.BlockSpec((8, HD), lambda i: (0, 0)),
        pl.BlockSpec((8, HD), lambda i: (0, 0)),
    ]
    out_specs = [
        pl.BlockSpec((BLK, HD), lambda i: (i, 0)),
        pl.BlockSpec((BLK, HD), lambda i: (i, 0)),
        pl.BlockSpec((GG, HD), lambda i: (0, 0)),
    ]
    return pl.pallas_call(
        body,
        grid=(grid,),
        in_specs=in_specs,
        out_specs=out_specs,
        out_shape=[jax.ShapeDtypeStruct((n_pad, HD), jnp.float32),
                   jax.ShapeDtypeStruct((n_pad, HD), jnp.bfloat16),
                   jax.ShapeDtypeStruct((GG, HD), jnp.float32)],
    )


def kernel(x, edge_index, batch, W1_0, b1_0, W2_0, b2_0, W1_1, b1_1, W2_1,
           b2_1, W1_2, b1_2, W2_2, b2_2):
    n, fin0 = x.shape
    e = edge_index.shape[1]
    n_pad = ((n + BLK - 1) // BLK) * BLK
    e_unit = NS * 16 * EB
    e_pad = ((e + e_unit - 1) // e_unit) * e_unit
    src = jnp.pad(edge_index[0], (0, e_pad - e),
                  constant_values=n).reshape(e_pad // EB, EB)
    dst = jnp.pad(edge_index[1], (0, e_pad - e),
                  constant_values=n).reshape(e_pad // EB, EB)
    h = jnp.pad(x, ((0, n_pad - n), (0, 0)))
    h_bf = h.astype(jnp.bfloat16)
    batch_p = jnp.pad(batch, (0, n_pad - n), constant_values=GG)
    batch3d = batch_p.reshape(n_pad // BLK, 1, BLK)
    zeros_tab = jnp.zeros((n_pad, 2, 128), jnp.bfloat16)

    params = [(W1_0, b1_0, W2_0, b2_0), (W1_1, b1_1, W2_1, b2_1),
              (W1_2, b1_2, W2_2, b2_2)]
    pooled_all = []
    for (W1, b1, W2, b2) in params:
        fin = W1.shape[0]
        edge_split = fin == CW
        if edge_split:
            a0, a1 = _edge_agg_fn(n_pad, e_pad, True, 8)(
                src, dst, zeros_tab, h_bf.reshape(n_pad, 2, 128))
        else:
            a0, a1 = _edge_agg_fn(n_pad, e_pad, False, 16)(
                src, dst, zeros_tab,
                lax.slice_in_dim(h_bf, 0, CW, axis=1).reshape(n_pad, 2, 128),
                lax.slice_in_dim(h_bf, CW, 2 * CW,
                                 axis=1).reshape(n_pad, 2, 128))
        a0 = a0.reshape(n_pad, CW)
        a1 = a1.reshape(n_pad, CW)
        b1b = jnp.broadcast_to(b1.reshape(1, HD), (8, HD))
        b2b = jnp.broadcast_to(b2.reshape(1, HD), (8, HD))
        h, h_bf, pooled = _mlp_pool_fn(fin, edge_split, n_pad)(
            h, a0, a1, batch3d, W1.astype(jnp.bfloat16),
            W2.astype(jnp.bfloat16), b1b, b2b)
        pooled_all.append(pooled)

    z = jnp.concatenate(pooled_all, axis=1)
    center = jnp.zeros((1, HD * len(params)), jnp.float32)
    return (z, center)


# speedup vs baseline: 4.2788x; 1.0077x over previous
import functools

import jax
import jax.numpy as jnp
from jax import lax
from jax.experimental import pallas as pl
from jax.experimental.pallas import tpu as pltpu
from jax.experimental.pallas import tpu_sc as plsc

NC = 2
NS = 16
CW = 256
EB = 128
GG = 64
HD = 512
BLK = 512
NBUF = 2
LOOK = 2


@functools.cache
def _edge_agg_fn(n_pad, e_pad, edge_split, kb):
    e_core = e_pad // NC if edge_split else e_pad
    per_sub = e_core // NS
    nb_super = per_sub // (kb * EB)
    idx_rows_per_sub = per_sub // EB
    rows_per_sub = n_pad // NS
    mesh = plsc.VectorSubcoreMesh(core_axis_name="c", subcore_axis_name="s",
                                  num_cores=NC, num_subcores=NS)

    def body(src_hbm, dst_hbm, zeros_hbm, *refs):
        if edge_split:
            h_refs = (refs[0], refs[0])
            out_refs = refs[1:3]
            rest = refs[3:]
        else:
            h_refs = refs[0:2]
            out_refs = refs[2:4]
            rest = refs[4:]
        src_i, dst_i = rest[0], rest[1]
        rows = rest[2:2 + NBUF]
        accum, gsem, ssem = rest[2 + NBUF:]
        c = lax.axis_index("c")
        s = lax.axis_index("s")
        r0 = s * rows_per_sub
        core_idx0 = c * (idx_rows_per_sub * NS) if edge_split else 0

        def process(h_hbm, out_hbm):
            pltpu.sync_copy(zeros_hbm.at[pl.ds(r0, rows_per_sub)],
                            accum.at[pl.ds(r0, rows_per_sub)])
            plsc.subcore_barrier()

            def step(b, carry):
                ri = core_idx0 + s * idx_rows_per_sub + b * kb
                pltpu.sync_copy(src_hbm.at[pl.ds(ri, kb)], src_i)
                pltpu.sync_copy(dst_hbm.at[pl.ds(ri, kb)], dst_i)
                gd = [None] * kb
                sd = [None] * kb
                for j in range(LOOK):
                    gd[j] = pltpu.async_copy(
                        h_hbm.at[src_i.at[j]], rows[j % NBUF], gsem)
                for j in range(kb):
                    gd[j].wait()
                    sd[j] = pltpu.async_copy(
                        rows[j % NBUF], accum.at[dst_i.at[j]], ssem,
                        add=True)
                    nxt = j + LOOK
                    if nxt < kb:
                        if nxt - NBUF >= 0:
                            sd[nxt - NBUF].wait()
                        gd[nxt] = pltpu.async_copy(
                            h_hbm.at[src_i.at[nxt]], rows[nxt % NBUF], gsem)
                for j in range(kb - NBUF, kb):
                    sd[j].wait()
                return carry

            lax.fori_loop(0, nb_super, step, 0)
            plsc.subcore_barrier()
            pltpu.sync_copy(accum.at[pl.ds(r0, rows_per_sub)],
                            out_hbm.at[pl.ds(r0, rows_per_sub)])

        @pl.when(c == 0)
        def _():
            process(h_refs[0], out_refs[0])

        @pl.when(c == 1)
        def _():
            process(h_refs[1], out_refs[1])

    return pl.kernel(
        body,
        out_type=[jax.ShapeDtypeStruct((n_pad, 2, 128), jnp.bfloat16)] * 2,
        mesh=mesh,
        compiler_params=pltpu.CompilerParams(use_tc_tiling_on_sc=False),
        scratch_types=(
            [pltpu.VMEM((kb, EB), jnp.int32)] * 2
            + [pltpu.VMEM((EB, 2, 128), jnp.bfloat16)] * NBUF
            + [pltpu.VMEM_SHARED((n_pad, 2, 128), jnp.bfloat16),
               pltpu.SemaphoreType.DMA,
               pltpu.SemaphoreType.DMA]
        ),
    )


@functools.cache
def _mlp_pool_fn(fin, agg_sum, n_pad):
    grid = n_pad // BLK

    def body(*refs):
        (h_ref, a0_ref, a1_ref, batch_ref, w1_ref, w2_ref, b1_ref, b2_ref,
         hout_ref, hbf_ref, pooled_ref) = refs
        i = pl.program_id(0)
        a0 = a0_ref[...].astype(jnp.float32)
        a1 = a1_ref[...].astype(jnp.float32)
        if agg_sum:
            agg = a0 + a1
        else:
            agg = jnp.concatenate([a0, a1], axis=1)
        hin = (h_ref[...] + agg).astype(jnp.bfloat16)
        t = jnp.dot(hin, w1_ref[...], preferred_element_type=jnp.float32)
        t = jnp.maximum(t + b1_ref[0:1, :], 0.0).astype(jnp.bfloat16)
        o = jnp.dot(t, w2_ref[...], preferred_element_type=jnp.float32)
        o = jnp.maximum(o + b2_ref[0:1, :], 0.0)
        b = batch_ref[0, 0, :]
        o = jnp.where(b.reshape(BLK, 1) < GG, o, 0.0)
        hout_ref[...] = o
        hbf_ref[...] = o.astype(jnp.bfloat16)
        onehot = (jax.lax.broadcasted_iota(jnp.int32, (GG, BLK), 0)
                  == b.reshape(1, BLK)).astype(jnp.float32)
        ppart = jnp.dot(onehot, o, preferred_element_type=jnp.float32)

        @pl.when(i == 0)
        def _():
            pooled_ref[...] = ppart

        @pl.when(i > 0)
        def _():
            pooled_ref[...] += ppart

    in_specs = [
        pl.BlockSpec((BLK, fin), lambda i: (i, 0)),
        pl.BlockSpec((BLK, CW), lambda i: (i, 0)),
        pl.BlockSpec((BLK, CW), lambda i: (i, 0)),
        pl.BlockSpec((1, 1, BLK), lambda i: (i, 0, 0)),
        pl.BlockSpec((fin, HD), lambda i: (0, 0)),
        pl.BlockSpec((HD, HD), lambda i: (0, 0)),
        pl.BlockSpec((8, HD), lambda i: (0, 0)),
        pl.BlockSpec((8, HD), lambda i: (0, 0)),
    ]
    out_specs = [
        pl.BlockSpec((BLK, HD), lambda i: (i, 0)),
        pl.BlockSpec((BLK, HD), lambda i: (i, 0)),
        pl.BlockSpec((GG, HD), lambda i: (0, 0)),
    ]
    return pl.pallas_call(
        body,
        grid=(grid,),
        in_specs=in_specs,
        out_specs=out_specs,
        out_shape=[jax.ShapeDtypeStruct((n_pad, HD), jnp.float32),
                   jax.ShapeDtypeStruct((n_pad, HD), jnp.bfloat16),
                   jax.ShapeDtypeStruct((GG, HD), jnp.float32)],
    )


def kernel(x, edge_index, batch, W1_0, b1_0, W2_0, b2_0, W1_1, b1_1, W2_1,
           b2_1, W1_2, b1_2, W2_2, b2_2):
    n, fin0 = x.shape
    e = edge_index.shape[1]
    n_pad = ((n + BLK - 1) // BLK) * BLK
    e_unit = NS * NC * 20 * EB
    e_pad = ((e + e_unit - 1) // e_unit) * e_unit
    src = jnp.pad(edge_index[0], (0, e_pad - e),
                  constant_values=n).reshape(e_pad // EB, EB)
    dst = jnp.pad(edge_index[1], (0, e_pad - e),
                  constant_values=n).reshape(e_pad // EB, EB)
    h = jnp.pad(x, ((0, n_pad - n), (0, 0)))
    h_bf = h.astype(jnp.bfloat16)
    batch_p = jnp.pad(batch, (0, n_pad - n), constant_values=GG)
    batch3d = batch_p.reshape(n_pad // BLK, 1, BLK)
    zeros_tab = jnp.zeros((n_pad, 2, 128), jnp.bfloat16)

    params = [(W1_0, b1_0, W2_0, b2_0), (W1_1, b1_1, W2_1, b2_1),
              (W1_2, b1_2, W2_2, b2_2)]
    pooled_all = []
    for (W1, b1, W2, b2) in params:
        fin = W1.shape[0]
        edge_split = fin == CW
        if edge_split:
            a0, a1 = _edge_agg_fn(n_pad, e_pad, True, 20)(
                src, dst, zeros_tab, h_bf.reshape(n_pad, 2, 128))
        else:
            a0, a1 = _edge_agg_fn(n_pad, e_pad, False, 20)(
                src, dst, zeros_tab,
                lax.slice_in_dim(h_bf, 0, CW, axis=1).reshape(n_pad, 2, 128),
                lax.slice_in_dim(h_bf, CW, 2 * CW,
                                 axis=1).reshape(n_pad, 2, 128))
        a0 = a0.reshape(n_pad, CW)
        a1 = a1.reshape(n_pad, CW)
        b1b = jnp.broadcast_to(b1.reshape(1, HD), (8, HD))
        b2b = jnp.broadcast_to(b2.reshape(1, HD), (8, HD))
        h, h_bf, pooled = _mlp_pool_fn(fin, edge_split, n_pad)(
            h, a0, a1, batch3d, W1.astype(jnp.bfloat16),
            W2.astype(jnp.bfloat16), b1b, b2b)
        pooled_all.append(pooled)

    z = jnp.concatenate(pooled_all, axis=1)
    center = jnp.zeros((1, HD * len(params)), jnp.float32)
    return (z, center)


# speedup vs baseline: 4.3113x; 1.0076x over previous
import functools

import jax
import jax.numpy as jnp
from jax import lax
from jax.experimental import pallas as pl
from jax.experimental.pallas import tpu as pltpu
from jax.experimental.pallas import tpu_sc as plsc

NC = 2
NS = 16
CW = 256
EB = 128
GG = 64
HD = 512
BLK = 512
NBUF = 2
LOOK = 2


@functools.cache
def _edge_agg_fn(n_pad, e_pad, edge_split, kb):
    e_core = e_pad // NC if edge_split else e_pad
    per_sub = e_core // NS
    nb_super = per_sub // (kb * EB)
    idx_rows_per_sub = per_sub // EB
    rows_per_sub = n_pad // NS
    mesh = plsc.VectorSubcoreMesh(core_axis_name="c", subcore_axis_name="s",
                                  num_cores=NC, num_subcores=NS)

    def body(src_hbm, dst_hbm, zeros_hbm, *refs):
        if edge_split:
            h_refs = (refs[0], refs[0])
            out_refs = refs[1:3]
            rest = refs[3:]
        else:
            h_refs = refs[0:2]
            out_refs = refs[2:4]
            rest = refs[4:]
        src_i, dst_i = rest[0], rest[1]
        rows = rest[2:2 + NBUF]
        accum, gsem, ssem = rest[2 + NBUF:]
        c = lax.axis_index("c")
        s = lax.axis_index("s")
        r0 = s * rows_per_sub
        core_idx0 = c * (idx_rows_per_sub * NS) if edge_split else 0

        def process(h_hbm, out_hbm):
            pltpu.sync_copy(zeros_hbm.at[pl.ds(r0, rows_per_sub)],
                            accum.at[pl.ds(r0, rows_per_sub)])
            plsc.subcore_barrier()

            def step(b, carry):
                ri = core_idx0 + s * idx_rows_per_sub + b * kb
                pltpu.sync_copy(src_hbm.at[pl.ds(ri, kb)], src_i)
                pltpu.sync_copy(dst_hbm.at[pl.ds(ri, kb)], dst_i)
                gd = [None] * kb
                sd = [None] * kb
                for j in range(LOOK):
                    gd[j] = pltpu.async_copy(
                        h_hbm.at[src_i.at[j]], rows[j % NBUF], gsem)
                for j in range(kb):
                    gd[j].wait()
                    sd[j] = pltpu.async_copy(
                        rows[j % NBUF], accum.at[dst_i.at[j]], ssem,
                        add=True)
                    nxt = j + LOOK
                    if nxt < kb:
                        if nxt - NBUF >= 0:
                            sd[nxt - NBUF].wait()
                        gd[nxt] = pltpu.async_copy(
                            h_hbm.at[src_i.at[nxt]], rows[nxt % NBUF], gsem)
                for j in range(kb - NBUF, kb):
                    sd[j].wait()
                return carry

            lax.fori_loop(0, nb_super, step, 0)
            plsc.subcore_barrier()
            pltpu.sync_copy(accum.at[pl.ds(r0, rows_per_sub)],
                            out_hbm.at[pl.ds(r0, rows_per_sub)])

        @pl.when(c == 0)
        def _():
            process(h_refs[0], out_refs[0])

        @pl.when(c == 1)
        def _():
            process(h_refs[1], out_refs[1])

    return pl.kernel(
        body,
        out_type=[jax.ShapeDtypeStruct((n_pad, 2, 128), jnp.bfloat16)] * 2,
        mesh=mesh,
        compiler_params=pltpu.CompilerParams(use_tc_tiling_on_sc=False),
        scratch_types=(
            [pltpu.VMEM((kb, EB), jnp.int32)] * 2
            + [pltpu.VMEM((EB, 2, 128), jnp.bfloat16)] * NBUF
            + [pltpu.VMEM_SHARED((n_pad, 2, 128), jnp.bfloat16),
               pltpu.SemaphoreType.DMA,
               pltpu.SemaphoreType.DMA]
        ),
    )


@functools.cache
def _mlp_pool_fn(fin, agg_sum, n_pad):
    grid = n_pad // BLK

    def body(*refs):
        (h_ref, a0_ref, a1_ref, batch_ref, w1_ref, w2_ref, b1_ref, b2_ref,
         hout_ref, hbf_ref, pooled_ref) = refs
        i = pl.program_id(0)
        a0 = a0_ref[...].astype(jnp.float32)
        a1 = a1_ref[...].astype(jnp.float32)
        if agg_sum:
            agg = a0 + a1
        else:
            agg = jnp.concatenate([a0, a1], axis=1)
        hin = (h_ref[...] + agg).astype(jnp.bfloat16)
        t = jnp.dot(hin, w1_ref[...], preferred_element_type=jnp.float32)
        t = jnp.maximum(t + b1_ref[0:1, :], 0.0).astype(jnp.bfloat16)
        o = jnp.dot(t, w2_ref[...], preferred_element_type=jnp.float32)
        o = jnp.maximum(o + b2_ref[0:1, :], 0.0)
        b = batch_ref[0, 0, :]
        o = jnp.where(b.reshape(BLK, 1) < GG, o, 0.0)
        hout_ref[...] = o
        hbf_ref[...] = o.astype(jnp.bfloat16)
        onehot = (jax.lax.broadcasted_iota(jnp.int32, (GG, BLK), 0)
                  == b.reshape(1, BLK)).astype(jnp.float32)
        ppart = jnp.dot(onehot, o, preferred_element_type=jnp.float32)

        @pl.when(i == 0)
        def _():
            pooled_ref[...] = ppart

        @pl.when(i > 0)
        def _():
            pooled_ref[...] += ppart

    in_specs = [
        pl.BlockSpec((BLK, fin), lambda i: (i, 0)),
        pl.BlockSpec((BLK, CW), lambda i: (i, 0)),
        pl.BlockSpec((BLK, CW), lambda i: (i, 0)),
        pl.BlockSpec((1, 1, BLK), lambda i: (i, 0, 0)),
        pl.BlockSpec((fin, HD), lambda i: (0, 0)),
        pl.BlockSpec((HD, HD), lambda i: (0, 0)),
        pl.BlockSpec((8, HD), lambda i: (0, 0)),
        pl.BlockSpec((8, HD), lambda i: (0, 0)),
    ]
    out_specs = [
        pl.BlockSpec((BLK, HD), lambda i: (i, 0)),
        pl.BlockSpec((BLK, HD), lambda i: (i, 0)),
        pl.BlockSpec((GG, HD), lambda i: (0, 0)),
    ]
    return pl.pallas_call(
        body,
        grid=(grid,),
        in_specs=in_specs,
        out_specs=out_specs,
        out_shape=[jax.ShapeDtypeStruct((n_pad, HD), jnp.float32),
                   jax.ShapeDtypeStruct((n_pad, HD), jnp.bfloat16),
                   jax.ShapeDtypeStruct((GG, HD), jnp.float32)],
    )


def kernel(x, edge_index, batch, W1_0, b1_0, W2_0, b2_0, W1_1, b1_1, W2_1,
           b2_1, W1_2, b1_2, W2_2, b2_2):
    n, fin0 = x.shape
    e = edge_index.shape[1]
    n_pad = ((n + BLK - 1) // BLK) * BLK
    e_unit = NS * NC * 40 * EB
    e_pad = ((e + e_unit - 1) // e_unit) * e_unit
    src = jnp.pad(edge_index[0], (0, e_pad - e),
                  constant_values=n).reshape(e_pad // EB, EB)
    dst = jnp.pad(edge_index[1], (0, e_pad - e),
                  constant_values=n).reshape(e_pad // EB, EB)
    h = jnp.pad(x, ((0, n_pad - n), (0, 0)))
    h_bf = h.astype(jnp.bfloat16)
    batch_p = jnp.pad(batch, (0, n_pad - n), constant_values=GG)
    batch3d = batch_p.reshape(n_pad // BLK, 1, BLK)
    zeros_tab = jnp.zeros((n_pad, 2, 128), jnp.bfloat16)

    params = [(W1_0, b1_0, W2_0, b2_0), (W1_1, b1_1, W2_1, b2_1),
              (W1_2, b1_2, W2_2, b2_2)]
    pooled_all = []
    for (W1, b1, W2, b2) in params:
        fin = W1.shape[0]
        edge_split = fin == CW
        if edge_split:
            a0, a1 = _edge_agg_fn(n_pad, e_pad, True, 40)(
                src, dst, zeros_tab, h_bf.reshape(n_pad, 2, 128))
        else:
            a0, a1 = _edge_agg_fn(n_pad, e_pad, False, 40)(
                src, dst, zeros_tab,
                lax.slice_in_dim(h_bf, 0, CW, axis=1).reshape(n_pad, 2, 128),
                lax.slice_in_dim(h_bf, CW, 2 * CW,
                                 axis=1).reshape(n_pad, 2, 128))
        a0 = a0.reshape(n_pad, CW)
        a1 = a1.reshape(n_pad, CW)
        b1b = jnp.broadcast_to(b1.reshape(1, HD), (8, HD))
        b2b = jnp.broadcast_to(b2.reshape(1, HD), (8, HD))
        h, h_bf, pooled = _mlp_pool_fn(fin, edge_split, n_pad)(
            h, a0, a1, batch3d, W1.astype(jnp.bfloat16),
            W2.astype(jnp.bfloat16), b1b, b2b)
        pooled_all.append(pooled)

    z = jnp.concatenate(pooled_all, axis=1)
    center = jnp.zeros((1, HD * len(params)), jnp.float32)
    return (z, center)
